# Initial kernel scaffold; baseline (speedup 1.0000x reference)
import jax
import jax.numpy as jnp
from jax.experimental import pallas as pl


def kernel(image_embeddings, text_embeddings, modal_table):
    raise NotImplementedError("write your pallas kernel here")


# speedup vs baseline: 3.4458x; 3.4458x over previous
import jax
import jax.numpy as jnp
from jax.experimental import pallas as pl

_BLOCK_ROWS = 1024


def _add_row_kernel(x_ref, row_ref, o_ref):
    o_ref[...] = x_ref[...] + row_ref[...]


def _broadcast_add(x2d, row):
    n, d = x2d.shape
    return pl.pallas_call(
        _add_row_kernel,
        grid=(n // _BLOCK_ROWS,),
        in_specs=[
            pl.BlockSpec((_BLOCK_ROWS, d), lambda i: (i, 0)),
            pl.BlockSpec((1, d), lambda i: (0, 0)),
        ],
        out_specs=pl.BlockSpec((_BLOCK_ROWS, d), lambda i: (i, 0)),
        out_shape=jax.ShapeDtypeStruct((n, d), x2d.dtype),
    )(x2d, row)


def kernel(image_embeddings, text_embeddings, modal_table):
    b, li, d = image_embeddings.shape
    lt = text_embeddings.shape[1]
    img = _broadcast_add(image_embeddings.reshape(b * li, d), modal_table[0:1, :])
    txt = _broadcast_add(text_embeddings.reshape(b * lt, d), modal_table[1:2, :])
    return img.reshape(b, li, d), txt.reshape(b, lt, d)


# speedup vs baseline: 3.5754x; 1.0376x over previous
import jax
import jax.numpy as jnp
from jax.experimental import pallas as pl

_BLOCK_ROWS = 2048


def _add_row_kernel(x_ref, row_ref, o_ref):
    o_ref[...] = x_ref[...] + row_ref[...]


def _broadcast_add(x2d, row):
    n, d = x2d.shape
    return pl.pallas_call(
        _add_row_kernel,
        grid=(n // _BLOCK_ROWS,),
        in_specs=[
            pl.BlockSpec((_BLOCK_ROWS, d), lambda i: (i, 0)),
            pl.BlockSpec((1, d), lambda i: (0, 0)),
        ],
        out_specs=pl.BlockSpec((_BLOCK_ROWS, d), lambda i: (i, 0)),
        out_shape=jax.ShapeDtypeStruct((n, d), x2d.dtype),
    )(x2d, row)


def kernel(image_embeddings, text_embeddings, modal_table):
    b, li, d = image_embeddings.shape
    lt = text_embeddings.shape[1]
    img = _broadcast_add(image_embeddings.reshape(b * li, d), modal_table[0:1, :])
    txt = _broadcast_add(text_embeddings.reshape(b * lt, d), modal_table[1:2, :])
    return img.reshape(b, li, d), txt.reshape(b, lt, d)


# speedup vs baseline: 3.6206x; 1.0127x over previous
import jax
import jax.numpy as jnp
from jax.experimental import pallas as pl

_BLOCK_ROWS = 4096


def _add_row_kernel(x_ref, row_ref, o_ref):
    o_ref[...] = x_ref[...] + row_ref[...]


def _broadcast_add(x2d, row):
    n, d = x2d.shape
    return pl.pallas_call(
        _add_row_kernel,
        grid=(n // _BLOCK_ROWS,),
        in_specs=[
            pl.BlockSpec((_BLOCK_ROWS, d), lambda i: (i, 0)),
            pl.BlockSpec((1, d), lambda i: (0, 0)),
        ],
        out_specs=pl.BlockSpec((_BLOCK_ROWS, d), lambda i: (i, 0)),
        out_shape=jax.ShapeDtypeStruct((n, d), x2d.dtype),
    )(x2d, row)


def kernel(image_embeddings, text_embeddings, modal_table):
    b, li, d = image_embeddings.shape
    lt = text_embeddings.shape[1]
    img = _broadcast_add(image_embeddings.reshape(b * li, d), modal_table[0:1, :])
    txt = _broadcast_add(text_embeddings.reshape(b * lt, d), modal_table[1:2, :])
    return img.reshape(b, li, d), txt.reshape(b, lt, d)


# speedup vs baseline: 3.6352x; 1.0040x over previous
import jax
import jax.numpy as jnp
from jax.experimental import pallas as pl

_GRID = 32


def _add_rows_kernel(img_ref, txt_ref, tab_ref, img_out_ref, txt_out_ref):
    img_out_ref[...] = img_ref[...] + tab_ref[0:1, :]
    txt_out_ref[...] = txt_ref[...] + tab_ref[1:2, :]


def kernel(image_embeddings, text_embeddings, modal_table):
    b, li, d = image_embeddings.shape
    lt = text_embeddings.shape[1]
    ni, nt = b * li, b * lt
    bi, bt = ni // _GRID, nt // _GRID
    img2d = image_embeddings.reshape(ni, d)
    txt2d = text_embeddings.reshape(nt, d)
    img, txt = pl.pallas_call(
        _add_rows_kernel,
        grid=(_GRID,),
        in_specs=[
            pl.BlockSpec((bi, d), lambda i: (i, 0)),
            pl.BlockSpec((bt, d), lambda i: (i, 0)),
            pl.BlockSpec((2, d), lambda i: (0, 0)),
        ],
        out_specs=[
            pl.BlockSpec((bi, d), lambda i: (i, 0)),
            pl.BlockSpec((bt, d), lambda i: (i, 0)),
        ],
        out_shape=[
            jax.ShapeDtypeStruct((ni, d), img2d.dtype),
            jax.ShapeDtypeStruct((nt, d), txt2d.dtype),
        ],
    )(img2d, txt2d, modal_table)
    return img.reshape(b, li, d), txt.reshape(b, lt, d)


# speedup vs baseline: 3.6600x; 1.0068x over previous
import jax
import jax.numpy as jnp
from jax.experimental import pallas as pl

_GRID = 16


def _add_rows_kernel(img_ref, txt_ref, tab_ref, img_out_ref, txt_out_ref):
    img_out_ref[...] = img_ref[...] + tab_ref[0:1, :]
    txt_out_ref[...] = txt_ref[...] + tab_ref[1:2, :]


def kernel(image_embeddings, text_embeddings, modal_table):
    b, li, d = image_embeddings.shape
    lt = text_embeddings.shape[1]
    ni, nt = b * li, b * lt
    bi, bt = ni // _GRID, nt // _GRID
    img2d = image_embeddings.reshape(ni, d)
    txt2d = text_embeddings.reshape(nt, d)
    img, txt = pl.pallas_call(
        _add_rows_kernel,
        grid=(_GRID,),
        in_specs=[
            pl.BlockSpec((bi, d), lambda i: (i, 0)),
            pl.BlockSpec((bt, d), lambda i: (i, 0)),
            pl.BlockSpec((2, d), lambda i: (0, 0)),
        ],
        out_specs=[
            pl.BlockSpec((bi, d), lambda i: (i, 0)),
            pl.BlockSpec((bt, d), lambda i: (i, 0)),
        ],
        out_shape=[
            jax.ShapeDtypeStruct((ni, d), img2d.dtype),
            jax.ShapeDtypeStruct((nt, d), txt2d.dtype),
        ],
    )(img2d, txt2d, modal_table)
    return img.reshape(b, li, d), txt.reshape(b, lt, d)
